# Initial kernel scaffold; baseline (speedup 1.0000x reference)
import functools

import jax
import jax.numpy as jnp
from jax import lax
from jax.experimental import pallas as pl
from jax.experimental.pallas import tpu as pltpu
from jax.experimental.pallas import tpu_sc as plsc

N_NODES = 10000
D = 128
N_EDGES = 320000

NC = 2
NS = 16
L = 16
NW = NC * NS

CH = 128
NCHUNK = 80
E_PAD = NW * NCHUNK * CH
N_PAD = 10240
ROWS_PER_SUB = N_PAD // NS
ZROWS = 64


def _make_sc_agg(with_deg: bool):
    mesh = plsc.VectorSubcoreMesh(core_axis_name="c", subcore_axis_name="s")
    out_type = [
        jax.ShapeDtypeStruct((NC, N_PAD, D), jnp.float32),
        jax.ShapeDtypeStruct((NC, N_PAD, L), jnp.float32),
    ]
    scratch_types = [
        pltpu.VMEM((NCHUNK, CH), jnp.int32),
        pltpu.VMEM((NCHUNK, CH), jnp.int32),
        pltpu.VMEM((CH, D), jnp.float32),
        pltpu.VMEM((CH, L), jnp.float32),
        pltpu.VMEM((ZROWS, D), jnp.float32),
        pltpu.VMEM((ZROWS, L), jnp.float32),
        pltpu.VMEM_SHARED((N_PAD, D), jnp.float32),
        pltpu.VMEM_SHARED((N_PAD, L), jnp.float32),
    ]

    @functools.partial(pl.kernel, mesh=mesh, out_type=out_type,
                       scratch_types=scratch_types)
    def sc_agg(table, srci, dsti, agg_out, deg_out,
               src_v, dst_v, rows_v, ones_v, zbuf, zdeg, acc_sh, deg_sh):
        cid = lax.axis_index("c")
        sid = lax.axis_index("s")
        wid = sid * NC + cid
        base = sid * ROWS_PER_SUB

        zeros16 = jnp.zeros((L,), jnp.float32)
        ones16 = jnp.ones((L,), jnp.float32)

        @pl.loop(0, ZROWS)
        def _(i):
            zdeg[i, :] = zeros16

            @pl.loop(0, D, step=L)
            def _(c):
                zbuf[i, pl.ds(c, L)] = zeros16

        @pl.loop(0, CH)
        def _(i):
            ones_v[i, :] = ones16

        @pl.loop(0, ROWS_PER_SUB, step=ZROWS)
        def _(r):
            pltpu.sync_copy(zbuf, acc_sh.at[pl.ds(base + r, ZROWS)])
            if with_deg:
                pltpu.sync_copy(zdeg, deg_sh.at[pl.ds(base + r, ZROWS)])

        pltpu.sync_copy(srci.at[wid], src_v)
        pltpu.sync_copy(dsti.at[wid], dst_v)

        plsc.subcore_barrier()

        @pl.loop(0, NCHUNK)
        def _(j):
            pltpu.sync_copy(table.at[src_v.at[j]], rows_v)
            pltpu.sync_copy(rows_v, acc_sh.at[dst_v.at[j]], add=True)
            if with_deg:
                pltpu.sync_copy(ones_v, deg_sh.at[dst_v.at[j]], add=True)

        plsc.subcore_barrier()

        pltpu.sync_copy(acc_sh.at[pl.ds(base, ROWS_PER_SUB)],
                        agg_out.at[cid, pl.ds(base, ROWS_PER_SUB)])
        pltpu.sync_copy(deg_sh.at[pl.ds(base, ROWS_PER_SUB)],
                        deg_out.at[cid, pl.ds(base, ROWS_PER_SUB)])

    return sc_agg


_sc_agg_deg = _make_sc_agg(with_deg=True)
_sc_agg_nodeg = _make_sc_agg(with_deg=False)

BR = 512


def _make_tc_layer(relu: bool):
    def body(p_ref, dp_ref, x_ref, wl_ref, bl_ref, wr_ref, o_ref):
        p = p_ref[0] + p_ref[1]
        deg = dp_ref[0, :, 0:1] + dp_ref[1, :, 0:1]
        mean = p / jnp.maximum(deg, 1.0)
        dn = (((1,), (1,)), ((), ()))
        acc = lax.dot_general(mean, wl_ref[...], dn,
                              precision=lax.Precision.HIGHEST,
                              preferred_element_type=jnp.float32)
        acc = acc + lax.dot_general(x_ref[...], wr_ref[...], dn,
                                    precision=lax.Precision.HIGHEST,
                                    preferred_element_type=jnp.float32)
        acc = acc + bl_ref[...]
        o_ref[...] = jnp.maximum(acc, 0.0) if relu else acc

    grid = (N_PAD // BR,)
    return pl.pallas_call(
        body,
        grid=grid,
        in_specs=[
            pl.BlockSpec((NC, BR, D), lambda i: (0, i, 0)),
            pl.BlockSpec((NC, BR, L), lambda i: (0, i, 0)),
            pl.BlockSpec((BR, D), lambda i: (i, 0)),
            pl.BlockSpec((D, D), lambda i: (0, 0)),
            pl.BlockSpec((1, D), lambda i: (0, 0)),
            pl.BlockSpec((D, D), lambda i: (0, 0)),
        ],
        out_specs=pl.BlockSpec((BR, D), lambda i: (i, 0)),
        out_shape=jax.ShapeDtypeStruct((N_PAD, D), jnp.float32),
    )


_tc_layer_relu = _make_tc_layer(relu=True)
_tc_layer_lin = _make_tc_layer(relu=False)


def kernel(x, edge_index, W1l, b1, W1r, W2l, b2, W2r):
    src = edge_index[0].astype(jnp.int32)
    dst = edge_index[1].astype(jnp.int32)
    npad = E_PAD - N_EDGES
    pad_dst = N_NODES + jnp.arange(npad, dtype=jnp.int32) % (N_PAD - N_NODES)
    src_p = jnp.concatenate([src, jnp.zeros((npad,), jnp.int32)])
    dst_p = jnp.concatenate([dst, pad_dst])
    srci = src_p.reshape(NW, NCHUNK, CH)
    dsti = dst_p.reshape(NW, NCHUNK, CH)

    x_pad = jnp.pad(x, ((0, N_PAD - N_NODES), (0, 0)))
    b1r = b1.reshape(1, D)
    b2r = b2.reshape(1, D)

    agg1, deg = _sc_agg_deg(x_pad, srci, dsti)
    h = _tc_layer_relu(agg1, deg, x_pad, W1l, b1r, W1r)
    agg2, _ = _sc_agg_nodeg(h, srci, dsti)
    out = _tc_layer_lin(agg2, deg, h, W2l, b2r, W2r)
    return out[:N_NODES]


# speedup vs baseline: 2.3069x; 2.3069x over previous
import functools

import jax
import jax.numpy as jnp
from jax import lax
from jax.experimental import pallas as pl
from jax.experimental.pallas import tpu as pltpu
from jax.experimental.pallas import tpu_sc as plsc

N_NODES = 10000
D = 128
N_EDGES = 320000

NC = 2
NS = 16

CH = 64
NCHUNK = 160
NW = NC * NS
E_PAD = NW * NCHUNK * CH
N_PAD = 10240
ROWS_PER_SUB = N_PAD // NS
ZROWS = 64

_sc_mesh = plsc.VectorSubcoreMesh(core_axis_name="c", subcore_axis_name="s")


@functools.partial(
    pl.kernel,
    mesh=_sc_mesh,
    out_type=jax.ShapeDtypeStruct((NC * N_PAD, D), jnp.float32),
    scratch_types=[
        pltpu.VMEM((CH,), jnp.int32),
        pltpu.VMEM((CH,), jnp.int32),
        pltpu.VMEM((CH, D), jnp.float32),
        pltpu.SemaphoreType.DMA,
        pltpu.VMEM_SHARED((N_PAD, D), jnp.float32),
    ],
)
def _sc_agg(table, srci, dsti, zrows, parts, srow, drow, rows_v, sem, acc_sh):
    cid = lax.axis_index("c")
    sid = lax.axis_index("s")
    wid = sid * NC + cid
    ebase = wid * (NCHUNK * CH)
    base = sid * ROWS_PER_SUB

    pltpu.sync_copy(zrows, rows_v)

    @pl.loop(0, ROWS_PER_SUB, step=ZROWS)
    def _(r):
        pltpu.sync_copy(rows_v, acc_sh.at[pl.ds(base + r, ZROWS)])

    plsc.subcore_barrier()

    @pl.loop(0, NCHUNK)
    def _(j):
        pltpu.sync_copy(srci.at[pl.ds(ebase + j * CH, CH)], srow)
        pltpu.sync_copy(dsti.at[pl.ds(ebase + j * CH, CH)], drow)
        pltpu.async_copy(table.at[srow], rows_v, sem).wait()
        pltpu.sync_copy(rows_v, acc_sh.at[drow], add=True)

    plsc.subcore_barrier()

    @pl.loop(0, ROWS_PER_SUB, step=ZROWS)
    def _(r):
        pltpu.sync_copy(acc_sh.at[pl.ds(base + r, ZROWS)], rows_v)
        pltpu.sync_copy(rows_v, parts.at[pl.ds(cid * N_PAD + base + r, ZROWS)])


@functools.partial(
    pl.kernel,
    mesh=_sc_mesh,
    out_type=jax.ShapeDtypeStruct((NC * N_PAD, D), jnp.float32),
    scratch_types=[
        pltpu.VMEM((CH,), jnp.int32),
        pltpu.VMEM((CH, D), jnp.float32),
        pltpu.VMEM_SHARED((N_PAD, D), jnp.float32),
    ],
)
def _sc_deg(dsti, zrows, ones_hbm, parts, drow, rows_v, acc_sh):
    cid = lax.axis_index("c")
    sid = lax.axis_index("s")
    wid = sid * NC + cid
    ebase = wid * (NCHUNK * CH)
    base = sid * ROWS_PER_SUB

    pltpu.sync_copy(zrows, rows_v)

    @pl.loop(0, ROWS_PER_SUB, step=ZROWS)
    def _(r):
        pltpu.sync_copy(rows_v, acc_sh.at[pl.ds(base + r, ZROWS)])

    pltpu.sync_copy(ones_hbm, rows_v)
    plsc.subcore_barrier()

    @pl.loop(0, NCHUNK)
    def _(j):
        pltpu.sync_copy(dsti.at[pl.ds(ebase + j * CH, CH)], drow)
        pltpu.sync_copy(rows_v, acc_sh.at[drow], add=True)

    plsc.subcore_barrier()

    @pl.loop(0, ROWS_PER_SUB, step=ZROWS)
    def _(r):
        pltpu.sync_copy(acc_sh.at[pl.ds(base + r, ZROWS)], rows_v)
        pltpu.sync_copy(rows_v, parts.at[pl.ds(cid * N_PAD + base + r, ZROWS)])


BR = 512


def _make_tc_layer(relu: bool):
    def tc_body(p_ref, dp_ref, t_ref, wl_ref, bl_ref, wr_ref, o_ref):
        p = p_ref[0] + p_ref[1]
        deg = dp_ref[0, :, 0:1] + dp_ref[1, :, 0:1]
        mean = p / jnp.maximum(deg, 1.0)
        dn = (((1,), (1,)), ((), ()))
        acc = lax.dot_general(mean, wl_ref[...], dn,
                              precision=lax.Precision.HIGHEST,
                              preferred_element_type=jnp.float32)
        acc = acc + lax.dot_general(t_ref[...], wr_ref[...], dn,
                                    precision=lax.Precision.HIGHEST,
                                    preferred_element_type=jnp.float32)
        acc = acc + bl_ref[...]
        o_ref[...] = jnp.maximum(acc, 0.0) if relu else acc

    return pl.pallas_call(
        tc_body,
        grid=(N_PAD // BR,),
        in_specs=[
            pl.BlockSpec((NC, BR, D), lambda i: (0, i, 0)),
            pl.BlockSpec((NC, BR, D), lambda i: (0, i, 0)),
            pl.BlockSpec((BR, D), lambda i: (i, 0)),
            pl.BlockSpec((D, D), lambda i: (0, 0)),
            pl.BlockSpec((1, D), lambda i: (0, 0)),
            pl.BlockSpec((D, D), lambda i: (0, 0)),
        ],
        out_specs=pl.BlockSpec((BR, D), lambda i: (i, 0)),
        out_shape=jax.ShapeDtypeStruct((N_PAD, D), jnp.float32),
    )


_tc_layer_relu = _make_tc_layer(relu=True)
_tc_layer_lin = _make_tc_layer(relu=False)


def kernel(x, edge_index, W1l, b1, W1r, W2l, b2, W2r):
    src = edge_index[0].astype(jnp.int32)
    dst = edge_index[1].astype(jnp.int32)
    npad = E_PAD - N_EDGES
    pad_dst = N_NODES + jnp.arange(npad, dtype=jnp.int32) % (N_PAD - N_NODES)
    srci = jnp.concatenate([src, jnp.zeros((npad,), jnp.int32)])
    dsti = jnp.concatenate([dst, pad_dst])

    x_pad = jnp.pad(x, ((0, N_PAD - N_NODES), (0, 0)))
    b1r = b1.reshape(1, D)
    b2r = b2.reshape(1, D)
    zrows = jnp.zeros((ZROWS, D), jnp.float32)
    ones_hbm = jnp.ones((CH, D), jnp.float32)

    def _stack(parts):
        return jnp.stack([parts[:N_PAD], parts[N_PAD:]])

    degp = _stack(_sc_deg(dsti, zrows, ones_hbm))
    agg1 = _stack(_sc_agg(x_pad, srci, dsti, zrows))
    h = _tc_layer_relu(agg1, degp, x_pad, W1l, b1r, W1r)
    agg2 = _stack(_sc_agg(h, srci, dsti, zrows))
    out = _tc_layer_lin(agg2, degp, h, W2l, b2r, W2r)
    return out[:N_NODES]


# speedup vs baseline: 3.3108x; 1.4351x over previous
import functools

import jax
import jax.numpy as jnp
from jax import lax
from jax.experimental import pallas as pl
from jax.experimental.pallas import tpu as pltpu
from jax.experimental.pallas import tpu_sc as plsc

N_NODES = 10000
D = 128
N_EDGES = 320000

NC = 2
NS = 16

CH = 128
NCHUNK = 80
NW = NC * NS
E_PAD = NW * NCHUNK * CH
N_PAD = 10240
ROWS_PER_SUB = N_PAD // NS

_sc_mesh = plsc.VectorSubcoreMesh(core_axis_name="c", subcore_axis_name="s")


@functools.partial(
    pl.kernel,
    mesh=_sc_mesh,
    out_type=jax.ShapeDtypeStruct((NC * N_PAD, D), jnp.float32),
    scratch_types=[
        pltpu.VMEM((CH,), jnp.int32),
        pltpu.VMEM((CH,), jnp.int32),
        pltpu.VMEM((CH,), jnp.int32),
        pltpu.VMEM((CH,), jnp.int32),
        pltpu.VMEM((CH, D), jnp.float32),
        pltpu.VMEM((CH, D), jnp.float32),
        pltpu.SemaphoreType.DMA,
        pltpu.SemaphoreType.DMA,
        pltpu.VMEM_SHARED((N_PAD, D), jnp.float32),
    ],
)
def _sc_agg(table, srci, dsti, zrows, parts, srow0, drow0, srow1, drow1,
            rows0, rows1, sem0, sem1, acc_sh):
    cid = lax.axis_index("c")
    sid = lax.axis_index("s")
    wid = sid * NC + cid
    ebase = wid * (NCHUNK * CH)
    base = sid * ROWS_PER_SUB

    pltpu.sync_copy(zrows, rows0)

    @pl.loop(0, ROWS_PER_SUB, step=CH)
    def _(r):
        pltpu.sync_copy(rows0, acc_sh.at[pl.ds(base + r, CH)])

    plsc.subcore_barrier()

    pltpu.sync_copy(srci.at[pl.ds(ebase, CH)], srow0)
    pltpu.sync_copy(dsti.at[pl.ds(ebase, CH)], drow0)
    pltpu.async_copy(table.at[srow0], rows0, sem0)

    @pl.loop(0, NCHUNK, step=2)
    def _(j):
        pltpu.sync_copy(srci.at[pl.ds(ebase + (j + 1) * CH, CH)], srow1)
        pltpu.sync_copy(dsti.at[pl.ds(ebase + (j + 1) * CH, CH)], drow1)
        pltpu.async_copy(table.at[srow1], rows1, sem1)
        pltpu.make_async_copy(table.at[srow0], rows0, sem0).wait()
        pltpu.sync_copy(rows0, acc_sh.at[drow0], add=True)

        @pl.when(j + 2 < NCHUNK)
        def _():
            pltpu.sync_copy(srci.at[pl.ds(ebase + (j + 2) * CH, CH)], srow0)
            pltpu.sync_copy(dsti.at[pl.ds(ebase + (j + 2) * CH, CH)], drow0)
            pltpu.async_copy(table.at[srow0], rows0, sem0)

        pltpu.make_async_copy(table.at[srow1], rows1, sem1).wait()
        pltpu.sync_copy(rows1, acc_sh.at[drow1], add=True)

    plsc.subcore_barrier()

    @pl.loop(0, ROWS_PER_SUB, step=CH)
    def _(r):
        pltpu.sync_copy(acc_sh.at[pl.ds(base + r, CH)], rows0)
        pltpu.sync_copy(rows0, parts.at[pl.ds(cid * N_PAD + base + r, CH)])


@functools.partial(
    pl.kernel,
    mesh=_sc_mesh,
    out_type=jax.ShapeDtypeStruct((NC * N_PAD, D), jnp.float32),
    scratch_types=[
        pltpu.VMEM((CH,), jnp.int32),
        pltpu.VMEM((CH, D), jnp.float32),
        pltpu.VMEM_SHARED((N_PAD, D), jnp.float32),
    ],
)
def _sc_deg(dsti, zrows, ones_hbm, parts, drow, rows_v, acc_sh):
    cid = lax.axis_index("c")
    sid = lax.axis_index("s")
    wid = sid * NC + cid
    ebase = wid * (NCHUNK * CH)
    base = sid * ROWS_PER_SUB

    pltpu.sync_copy(zrows, rows_v)

    @pl.loop(0, ROWS_PER_SUB, step=CH)
    def _(r):
        pltpu.sync_copy(rows_v, acc_sh.at[pl.ds(base + r, CH)])

    pltpu.sync_copy(ones_hbm, rows_v)
    plsc.subcore_barrier()

    @pl.loop(0, NCHUNK)
    def _(j):
        pltpu.sync_copy(dsti.at[pl.ds(ebase + j * CH, CH)], drow)
        pltpu.sync_copy(rows_v, acc_sh.at[drow], add=True)

    plsc.subcore_barrier()

    @pl.loop(0, ROWS_PER_SUB, step=CH)
    def _(r):
        pltpu.sync_copy(acc_sh.at[pl.ds(base + r, CH)], rows_v)
        pltpu.sync_copy(rows_v, parts.at[pl.ds(cid * N_PAD + base + r, CH)])


BR = 512


def _make_tc_layer(relu: bool):
    def tc_body(p_ref, dp_ref, t_ref, wl_ref, bl_ref, wr_ref, o_ref):
        p = p_ref[0] + p_ref[1]
        deg = dp_ref[0, :, 0:1] + dp_ref[1, :, 0:1]
        mean = p / jnp.maximum(deg, 1.0)
        dn = (((1,), (1,)), ((), ()))
        acc = lax.dot_general(mean, wl_ref[...], dn,
                              precision=lax.Precision.HIGHEST,
                              preferred_element_type=jnp.float32)
        acc = acc + lax.dot_general(t_ref[...], wr_ref[...], dn,
                                    precision=lax.Precision.HIGHEST,
                                    preferred_element_type=jnp.float32)
        acc = acc + bl_ref[...]
        o_ref[...] = jnp.maximum(acc, 0.0) if relu else acc

    return pl.pallas_call(
        tc_body,
        grid=(N_PAD // BR,),
        in_specs=[
            pl.BlockSpec((NC, BR, D), lambda i: (0, i, 0)),
            pl.BlockSpec((NC, BR, D), lambda i: (0, i, 0)),
            pl.BlockSpec((BR, D), lambda i: (i, 0)),
            pl.BlockSpec((D, D), lambda i: (0, 0)),
            pl.BlockSpec((1, D), lambda i: (0, 0)),
            pl.BlockSpec((D, D), lambda i: (0, 0)),
        ],
        out_specs=pl.BlockSpec((BR, D), lambda i: (i, 0)),
        out_shape=jax.ShapeDtypeStruct((N_PAD, D), jnp.float32),
    )


_tc_layer_relu = _make_tc_layer(relu=True)
_tc_layer_lin = _make_tc_layer(relu=False)


def kernel(x, edge_index, W1l, b1, W1r, W2l, b2, W2r):
    src = edge_index[0].astype(jnp.int32)
    dst = edge_index[1].astype(jnp.int32)
    npad = E_PAD - N_EDGES
    pad_dst = N_NODES + jnp.arange(npad, dtype=jnp.int32) % (N_PAD - N_NODES)
    srci = jnp.concatenate([src, jnp.zeros((npad,), jnp.int32)])
    dsti = jnp.concatenate([dst, pad_dst])

    x_pad = jnp.pad(x, ((0, N_PAD - N_NODES), (0, 0)))
    b1r = b1.reshape(1, D)
    b2r = b2.reshape(1, D)
    zrows = jnp.zeros((CH, D), jnp.float32)
    ones_hbm = jnp.ones((CH, D), jnp.float32)

    def _stack(parts):
        return jnp.stack([parts[:N_PAD], parts[N_PAD:]])

    degp = _stack(_sc_deg(dsti, zrows, ones_hbm))
    agg1 = _stack(_sc_agg(x_pad, srci, dsti, zrows))
    h = _tc_layer_relu(agg1, degp, x_pad, W1l, b1r, W1r)
    agg2 = _stack(_sc_agg(h, srci, dsti, zrows))
    out = _tc_layer_lin(agg2, degp, h, W2l, b2r, W2r)
    return out[:N_NODES]


# speedup vs baseline: 3.6153x; 1.0920x over previous
import functools

import jax
import jax.numpy as jnp
from jax import lax
from jax.experimental import pallas as pl
from jax.experimental.pallas import tpu as pltpu
from jax.experimental.pallas import tpu_sc as plsc

N_NODES = 10000
D = 128
N_EDGES = 320000

NC = 2
NS = 16

CH = 128
NCHUNK = 80
NW = NC * NS
E_PAD = NW * NCHUNK * CH
N_PAD = 10240
ROWS_PER_SUB = N_PAD // NS

_sc_mesh = plsc.VectorSubcoreMesh(core_axis_name="c", subcore_axis_name="s")


@functools.partial(
    pl.kernel,
    mesh=_sc_mesh,
    out_type=jax.ShapeDtypeStruct((NC * N_PAD, D), jnp.float32),
    scratch_types=[
        pltpu.VMEM((CH,), jnp.int32),
        pltpu.VMEM((CH,), jnp.int32),
        pltpu.VMEM((CH,), jnp.int32),
        pltpu.VMEM((CH,), jnp.int32),
        pltpu.VMEM((CH, D), jnp.float32),
        pltpu.VMEM((CH, D), jnp.float32),
        pltpu.SemaphoreType.DMA,
        pltpu.SemaphoreType.DMA,
        pltpu.VMEM_SHARED((N_PAD, D), jnp.float32),
    ],
)
def _sc_agg(table, srci, dsti, zrows, parts, srow0, drow0, srow1, drow1,
            rows0, rows1, sem0, sem1, acc_sh):
    cid = lax.axis_index("c")
    sid = lax.axis_index("s")
    wid = sid * NC + cid
    ebase = wid * (NCHUNK * CH)
    base = sid * ROWS_PER_SUB

    pltpu.sync_copy(zrows, rows0)

    @pl.loop(0, ROWS_PER_SUB, step=CH)
    def _(r):
        pltpu.sync_copy(rows0, acc_sh.at[pl.ds(base + r, CH)])

    plsc.subcore_barrier()

    pltpu.sync_copy(srci.at[pl.ds(ebase, CH)], srow0)
    pltpu.sync_copy(dsti.at[pl.ds(ebase, CH)], drow0)
    pltpu.async_copy(table.at[srow0], rows0, sem0)

    @pl.loop(0, NCHUNK, step=2)
    def _(j):
        pltpu.sync_copy(srci.at[pl.ds(ebase + (j + 1) * CH, CH)], srow1)
        pltpu.sync_copy(dsti.at[pl.ds(ebase + (j + 1) * CH, CH)], drow1)
        pltpu.async_copy(table.at[srow1], rows1, sem1)
        pltpu.make_async_copy(table.at[srow0], rows0, sem0).wait()
        pltpu.sync_copy(rows0, acc_sh.at[drow0], add=True)

        @pl.when(j + 2 < NCHUNK)
        def _():
            pltpu.sync_copy(srci.at[pl.ds(ebase + (j + 2) * CH, CH)], srow0)
            pltpu.sync_copy(dsti.at[pl.ds(ebase + (j + 2) * CH, CH)], drow0)
            pltpu.async_copy(table.at[srow0], rows0, sem0)

        pltpu.make_async_copy(table.at[srow1], rows1, sem1).wait()
        pltpu.sync_copy(rows1, acc_sh.at[drow1], add=True)

    plsc.subcore_barrier()

    @pl.loop(0, ROWS_PER_SUB, step=CH)
    def _(r):
        pltpu.sync_copy(acc_sh.at[pl.ds(base + r, CH)], rows0)
        pltpu.sync_copy(rows0, parts.at[pl.ds(cid * N_PAD + base + r, CH)])


@functools.partial(
    pl.kernel,
    mesh=_sc_mesh,
    out_type=[
        jax.ShapeDtypeStruct((NC * N_PAD, D), jnp.float32),
        jax.ShapeDtypeStruct((NC * N_PAD, D), jnp.float32),
    ],
    scratch_types=[
        pltpu.VMEM((CH,), jnp.int32),
        pltpu.VMEM((CH,), jnp.int32),
        pltpu.VMEM((CH,), jnp.int32),
        pltpu.VMEM((CH,), jnp.int32),
        pltpu.VMEM((CH, D), jnp.float32),
        pltpu.VMEM((CH, D), jnp.float32),
        pltpu.SemaphoreType.DMA,
        pltpu.SemaphoreType.DMA,
        pltpu.VMEM_SHARED((N_PAD, D), jnp.float32),
    ],
)
def _sc_agg_deg(table, srci, dsti, zrows, ones_hbm, parts, degp, srow0, drow0,
                srow1, drow1, rows0, rows1, sem0, sem1, acc_sh):
    cid = lax.axis_index("c")
    sid = lax.axis_index("s")
    wid = sid * NC + cid
    ebase = wid * (NCHUNK * CH)
    base = sid * ROWS_PER_SUB

    pltpu.sync_copy(zrows, rows0)

    @pl.loop(0, ROWS_PER_SUB, step=CH)
    def _(r):
        pltpu.sync_copy(rows0, acc_sh.at[pl.ds(base + r, CH)])

    pltpu.sync_copy(ones_hbm, rows0)
    plsc.subcore_barrier()

    @pl.loop(0, NCHUNK)
    def _(j):
        pltpu.sync_copy(dsti.at[pl.ds(ebase + j * CH, CH)], drow0)
        pltpu.sync_copy(rows0, acc_sh.at[drow0], add=True)

    plsc.subcore_barrier()

    @pl.loop(0, ROWS_PER_SUB, step=CH)
    def _(r):
        pltpu.sync_copy(acc_sh.at[pl.ds(base + r, CH)], rows0)
        pltpu.sync_copy(rows0, degp.at[pl.ds(cid * N_PAD + base + r, CH)])

    pltpu.sync_copy(zrows, rows0)

    @pl.loop(0, ROWS_PER_SUB, step=CH)
    def _(r):
        pltpu.sync_copy(rows0, acc_sh.at[pl.ds(base + r, CH)])

    plsc.subcore_barrier()

    pltpu.sync_copy(srci.at[pl.ds(ebase, CH)], srow0)
    pltpu.sync_copy(dsti.at[pl.ds(ebase, CH)], drow0)
    pltpu.async_copy(table.at[srow0], rows0, sem0)

    @pl.loop(0, NCHUNK, step=2)
    def _(j):
        pltpu.sync_copy(srci.at[pl.ds(ebase + (j + 1) * CH, CH)], srow1)
        pltpu.sync_copy(dsti.at[pl.ds(ebase + (j + 1) * CH, CH)], drow1)
        pltpu.async_copy(table.at[srow1], rows1, sem1)
        pltpu.make_async_copy(table.at[srow0], rows0, sem0).wait()
        pltpu.sync_copy(rows0, acc_sh.at[drow0], add=True)

        @pl.when(j + 2 < NCHUNK)
        def _():
            pltpu.sync_copy(srci.at[pl.ds(ebase + (j + 2) * CH, CH)], srow0)
            pltpu.sync_copy(dsti.at[pl.ds(ebase + (j + 2) * CH, CH)], drow0)
            pltpu.async_copy(table.at[srow0], rows0, sem0)

        pltpu.make_async_copy(table.at[srow1], rows1, sem1).wait()
        pltpu.sync_copy(rows1, acc_sh.at[drow1], add=True)

    plsc.subcore_barrier()

    @pl.loop(0, ROWS_PER_SUB, step=CH)
    def _(r):
        pltpu.sync_copy(acc_sh.at[pl.ds(base + r, CH)], rows0)
        pltpu.sync_copy(rows0, parts.at[pl.ds(cid * N_PAD + base + r, CH)])


BR = 512


def _make_tc_layer(relu: bool):
    def tc_body(p_ref, dp_ref, t_ref, wl_ref, bl_ref, wr_ref, o_ref):
        p = p_ref[0] + p_ref[1]
        deg = dp_ref[0, :, 0:1] + dp_ref[1, :, 0:1]
        mean = p / jnp.maximum(deg, 1.0)
        dn = (((1,), (1,)), ((), ()))
        acc = lax.dot_general(mean, wl_ref[...], dn,
                              precision=lax.Precision.HIGHEST,
                              preferred_element_type=jnp.float32)
        acc = acc + lax.dot_general(t_ref[...], wr_ref[...], dn,
                                    precision=lax.Precision.HIGHEST,
                                    preferred_element_type=jnp.float32)
        acc = acc + bl_ref[...]
        o_ref[...] = jnp.maximum(acc, 0.0) if relu else acc

    return pl.pallas_call(
        tc_body,
        grid=(N_PAD // BR,),
        in_specs=[
            pl.BlockSpec((NC, BR, D), lambda i: (0, i, 0)),
            pl.BlockSpec((NC, BR, D), lambda i: (0, i, 0)),
            pl.BlockSpec((BR, D), lambda i: (i, 0)),
            pl.BlockSpec((D, D), lambda i: (0, 0)),
            pl.BlockSpec((1, D), lambda i: (0, 0)),
            pl.BlockSpec((D, D), lambda i: (0, 0)),
        ],
        out_specs=pl.BlockSpec((BR, D), lambda i: (i, 0)),
        out_shape=jax.ShapeDtypeStruct((N_PAD, D), jnp.float32),
    )


_tc_layer_relu = _make_tc_layer(relu=True)
_tc_layer_lin = _make_tc_layer(relu=False)


def kernel(x, edge_index, W1l, b1, W1r, W2l, b2, W2r):
    src = edge_index[0].astype(jnp.int32)
    dst = edge_index[1].astype(jnp.int32)
    npad = E_PAD - N_EDGES
    pad_dst = N_NODES + jnp.arange(npad, dtype=jnp.int32) % (N_PAD - N_NODES)
    srci = jnp.concatenate([src, jnp.zeros((npad,), jnp.int32)])
    dsti = jnp.concatenate([dst, pad_dst])

    x_pad = jnp.pad(x, ((0, N_PAD - N_NODES), (0, 0)))
    b1r = b1.reshape(1, D)
    b2r = b2.reshape(1, D)
    zrows = jnp.zeros((CH, D), jnp.float32)
    ones_hbm = jnp.ones((CH, D), jnp.float32)

    def _stack(parts):
        return jnp.stack([parts[:N_PAD], parts[N_PAD:]])

    agg1_flat, degp_flat = _sc_agg_deg(x_pad, srci, dsti, zrows, ones_hbm)
    agg1, degp = _stack(agg1_flat), _stack(degp_flat)
    h = _tc_layer_relu(agg1, degp, x_pad, W1l, b1r, W1r)
    agg2 = _stack(_sc_agg(h, srci, dsti, zrows))
    out = _tc_layer_lin(agg2, degp, h, W2l, b2r, W2r)
    return out[:N_NODES]


# speedup vs baseline: 3.6295x; 1.0039x over previous
import functools

import jax
import jax.numpy as jnp
from jax import lax
from jax.experimental import pallas as pl
from jax.experimental.pallas import tpu as pltpu
from jax.experimental.pallas import tpu_sc as plsc

N_NODES = 10000
D = 128
N_EDGES = 320000

NC = 2
NS = 16

CH = 128
NCHUNK = 80
NW = NC * NS
E_PAD = NW * NCHUNK * CH
N_PAD = 10240
ROWS_PER_SUB = N_PAD // NS

_sc_mesh = plsc.VectorSubcoreMesh(core_axis_name="c", subcore_axis_name="s")


@functools.partial(
    pl.kernel,
    mesh=_sc_mesh,
    out_type=jax.ShapeDtypeStruct((NC * N_PAD, D), jnp.float32),
    scratch_types=[
        pltpu.VMEM((CH,), jnp.int32),
        pltpu.VMEM((CH,), jnp.int32),
        pltpu.VMEM((CH,), jnp.int32),
        pltpu.VMEM((CH,), jnp.int32),
        pltpu.VMEM((CH, D), jnp.float32),
        pltpu.VMEM((CH, D), jnp.float32),
        pltpu.SemaphoreType.DMA,
        pltpu.SemaphoreType.DMA,
        pltpu.VMEM_SHARED((N_PAD, D), jnp.float32),
    ],
)
def _sc_agg(table, srci, dsti, zrows, parts, srow0, drow0, srow1, drow1,
            rows0, rows1, sem0, sem1, acc_sh):
    cid = lax.axis_index("c")
    sid = lax.axis_index("s")
    wid = sid * NC + cid
    ebase = wid * (NCHUNK * CH)
    base = sid * ROWS_PER_SUB

    pltpu.sync_copy(zrows, rows0)

    @pl.loop(0, ROWS_PER_SUB, step=CH)
    def _(r):
        pltpu.sync_copy(rows0, acc_sh.at[pl.ds(base + r, CH)])

    plsc.subcore_barrier()

    pltpu.sync_copy(srci.at[pl.ds(ebase, CH)], srow0)
    pltpu.sync_copy(dsti.at[pl.ds(ebase, CH)], drow0)
    pltpu.async_copy(table.at[srow0], rows0, sem0)

    @pl.loop(0, NCHUNK, step=2)
    def _(j):
        pltpu.sync_copy(srci.at[pl.ds(ebase + (j + 1) * CH, CH)], srow1)
        pltpu.sync_copy(dsti.at[pl.ds(ebase + (j + 1) * CH, CH)], drow1)
        pltpu.async_copy(table.at[srow1], rows1, sem1)
        pltpu.make_async_copy(table.at[srow0], rows0, sem0).wait()
        pltpu.sync_copy(rows0, acc_sh.at[drow0], add=True)

        @pl.when(j + 2 < NCHUNK)
        def _():
            pltpu.sync_copy(srci.at[pl.ds(ebase + (j + 2) * CH, CH)], srow0)
            pltpu.sync_copy(dsti.at[pl.ds(ebase + (j + 2) * CH, CH)], drow0)
            pltpu.async_copy(table.at[srow0], rows0, sem0)

        pltpu.make_async_copy(table.at[srow1], rows1, sem1).wait()
        pltpu.sync_copy(rows1, acc_sh.at[drow1], add=True)

    plsc.subcore_barrier()

    @pl.loop(0, ROWS_PER_SUB, step=CH)
    def _(r):
        pltpu.sync_copy(acc_sh.at[pl.ds(base + r, CH)], rows0)
        pltpu.sync_copy(rows0, parts.at[pl.ds(cid * N_PAD + base + r, CH)])


@functools.partial(
    pl.kernel,
    mesh=_sc_mesh,
    out_type=[
        jax.ShapeDtypeStruct((NC * N_PAD, D), jnp.float32),
        jax.ShapeDtypeStruct((NC * N_PAD, D), jnp.float32),
    ],
    scratch_types=[
        pltpu.VMEM((CH,), jnp.int32),
        pltpu.VMEM((CH,), jnp.int32),
        pltpu.VMEM((CH,), jnp.int32),
        pltpu.VMEM((CH,), jnp.int32),
        pltpu.VMEM((CH, D), jnp.float32),
        pltpu.VMEM((CH, D), jnp.float32),
        pltpu.SemaphoreType.DMA,
        pltpu.SemaphoreType.DMA,
        pltpu.VMEM_SHARED((N_PAD, D), jnp.float32),
    ],
)
def _sc_agg_deg(table, srci, dsti, zrows, ones_hbm, parts, degp, srow0, drow0,
                srow1, drow1, rows0, rows1, sem0, sem1, acc_sh):
    cid = lax.axis_index("c")
    sid = lax.axis_index("s")
    wid = sid * NC + cid
    ebase = wid * (NCHUNK * CH)
    base = sid * ROWS_PER_SUB

    pltpu.sync_copy(zrows, rows0)

    @pl.loop(0, ROWS_PER_SUB, step=CH)
    def _(r):
        pltpu.sync_copy(rows0, acc_sh.at[pl.ds(base + r, CH)])

    pltpu.sync_copy(ones_hbm, rows0)
    plsc.subcore_barrier()

    @pl.loop(0, NCHUNK)
    def _(j):
        pltpu.sync_copy(dsti.at[pl.ds(ebase + j * CH, CH)], drow0)
        pltpu.sync_copy(rows0, acc_sh.at[drow0], add=True)

    plsc.subcore_barrier()

    @pl.loop(0, ROWS_PER_SUB, step=CH)
    def _(r):
        pltpu.sync_copy(acc_sh.at[pl.ds(base + r, CH)], rows0)
        pltpu.sync_copy(rows0, degp.at[pl.ds(cid * N_PAD + base + r, CH)])

    pltpu.sync_copy(zrows, rows0)

    @pl.loop(0, ROWS_PER_SUB, step=CH)
    def _(r):
        pltpu.sync_copy(rows0, acc_sh.at[pl.ds(base + r, CH)])

    plsc.subcore_barrier()

    pltpu.sync_copy(srci.at[pl.ds(ebase, CH)], srow0)
    pltpu.sync_copy(dsti.at[pl.ds(ebase, CH)], drow0)
    pltpu.async_copy(table.at[srow0], rows0, sem0)

    @pl.loop(0, NCHUNK, step=2)
    def _(j):
        pltpu.sync_copy(srci.at[pl.ds(ebase + (j + 1) * CH, CH)], srow1)
        pltpu.sync_copy(dsti.at[pl.ds(ebase + (j + 1) * CH, CH)], drow1)
        pltpu.async_copy(table.at[srow1], rows1, sem1)
        pltpu.make_async_copy(table.at[srow0], rows0, sem0).wait()
        pltpu.sync_copy(rows0, acc_sh.at[drow0], add=True)

        @pl.when(j + 2 < NCHUNK)
        def _():
            pltpu.sync_copy(srci.at[pl.ds(ebase + (j + 2) * CH, CH)], srow0)
            pltpu.sync_copy(dsti.at[pl.ds(ebase + (j + 2) * CH, CH)], drow0)
            pltpu.async_copy(table.at[srow0], rows0, sem0)

        pltpu.make_async_copy(table.at[srow1], rows1, sem1).wait()
        pltpu.sync_copy(rows1, acc_sh.at[drow1], add=True)

    plsc.subcore_barrier()

    @pl.loop(0, ROWS_PER_SUB, step=CH)
    def _(r):
        pltpu.sync_copy(acc_sh.at[pl.ds(base + r, CH)], rows0)
        pltpu.sync_copy(rows0, parts.at[pl.ds(cid * N_PAD + base + r, CH)])


BR = 512


_NB = N_PAD // BR


def _make_tc_layer(relu: bool):
    def tc_body(p0_ref, p1_ref, d0_ref, d1_ref, t_ref, wl_ref, bl_ref,
                wr_ref, o_ref):
        p = p0_ref[...] + p1_ref[...]
        deg = d0_ref[:, 0:1] + d1_ref[:, 0:1]
        mean = p / jnp.maximum(deg, 1.0)
        dn = (((1,), (1,)), ((), ()))
        acc = lax.dot_general(mean, wl_ref[...], dn,
                              precision=lax.Precision.HIGHEST,
                              preferred_element_type=jnp.float32)
        acc = acc + lax.dot_general(t_ref[...], wr_ref[...], dn,
                                    precision=lax.Precision.HIGHEST,
                                    preferred_element_type=jnp.float32)
        acc = acc + bl_ref[...]
        o_ref[...] = jnp.maximum(acc, 0.0) if relu else acc

    return pl.pallas_call(
        tc_body,
        grid=(_NB,),
        in_specs=[
            pl.BlockSpec((BR, D), lambda i: (i, 0)),
            pl.BlockSpec((BR, D), lambda i: (i + _NB, 0)),
            pl.BlockSpec((BR, D), lambda i: (i, 0)),
            pl.BlockSpec((BR, D), lambda i: (i + _NB, 0)),
            pl.BlockSpec((BR, D), lambda i: (i, 0)),
            pl.BlockSpec((D, D), lambda i: (0, 0)),
            pl.BlockSpec((1, D), lambda i: (0, 0)),
            pl.BlockSpec((D, D), lambda i: (0, 0)),
        ],
        out_specs=pl.BlockSpec((BR, D), lambda i: (i, 0)),
        out_shape=jax.ShapeDtypeStruct((N_PAD, D), jnp.float32),
    )


_tc_layer_relu = _make_tc_layer(relu=True)
_tc_layer_lin = _make_tc_layer(relu=False)


def kernel(x, edge_index, W1l, b1, W1r, W2l, b2, W2r):
    src = edge_index[0].astype(jnp.int32)
    dst = edge_index[1].astype(jnp.int32)
    npad = E_PAD - N_EDGES
    pad_dst = N_NODES + jnp.arange(npad, dtype=jnp.int32) % (N_PAD - N_NODES)
    srci = jnp.concatenate([src, jnp.zeros((npad,), jnp.int32)])
    dsti = jnp.concatenate([dst, pad_dst])

    x_pad = jnp.pad(x, ((0, N_PAD - N_NODES), (0, 0)))
    b1r = b1.reshape(1, D)
    b2r = b2.reshape(1, D)
    zrows = jnp.zeros((CH, D), jnp.float32)
    ones_hbm = jnp.ones((CH, D), jnp.float32)

    agg1, degp = _sc_agg_deg(x_pad, srci, dsti, zrows, ones_hbm)
    h = _tc_layer_relu(agg1, agg1, degp, degp, x_pad, W1l, b1r, W1r)
    agg2 = _sc_agg(h, srci, dsti, zrows)
    out = _tc_layer_lin(agg2, agg2, degp, degp, h, W2l, b2r, W2r)
    return out[:N_NODES]
